# baseline (speedup 1.0000x reference)
import jax
import jax.numpy as jnp
from jax.experimental import pallas as pl
from jax.experimental.pallas import tpu as pltpu

_N_CHUNKS = 16


def _copy_body(src_hbm, out_hbm, buf, in_sems, out_sems):
    rows, dim = src_hbm.shape
    ch = rows // _N_CHUNKS

    def in_copy(i):
        return pltpu.make_async_copy(
            src_hbm.at[pl.ds(i * ch, ch), :], buf.at[i], in_sems.at[i])

    def out_copy(i):
        return pltpu.make_async_copy(
            buf.at[i], out_hbm.at[pl.ds(i * ch, ch), :], out_sems.at[i])

    for i in range(_N_CHUNKS):
        in_copy(i).start()
    for i in range(_N_CHUNKS):
        in_copy(i).wait()
        out_copy(i).start()
    for i in range(_N_CHUNKS):
        out_copy(i).wait()


def kernel(input_ids, pos_table):
    seq_len = input_ids.shape[1]
    rows, dim = pos_table.shape
    assert seq_len == rows
    ch = rows // _N_CHUNKS
    return pl.pallas_call(
        _copy_body,
        in_specs=[pl.BlockSpec(memory_space=pltpu.MemorySpace.HBM)],
        out_specs=pl.BlockSpec(memory_space=pltpu.MemorySpace.HBM),
        out_shape=jax.ShapeDtypeStruct((seq_len, dim), pos_table.dtype),
        scratch_shapes=[
            pltpu.VMEM((_N_CHUNKS, ch, dim), pos_table.dtype),
            pltpu.SemaphoreType.DMA((_N_CHUNKS,)),
            pltpu.SemaphoreType.DMA((_N_CHUNKS,)),
        ],
    )(pos_table)
